# Initial kernel scaffold; baseline (speedup 1.0000x reference)
import jax
import jax.numpy as jnp
from jax.experimental import pallas as pl


def kernel(x, edge_index, batch_data, W1, b1, W2, b2, W3, b3, Wl, bl):
    raise NotImplementedError("write your pallas kernel here")


# speedup vs baseline: 15.9502x; 15.9502x over previous
import functools

import jax
import jax.numpy as jnp
from jax import lax
from jax.experimental import pallas as pl
from jax.experimental.pallas import tpu as pltpu
from jax.experimental.pallas import tpu_sc as plsc

N = 50000
E = 800000
H = 64
HH = 32
G = 512
NC, NS = 2, 16
NW = NC * NS

NP = 50176
EP = 802816
ER = EP // 128
RPT = NP // NS
TBS = 520
TRASH = NS * TBS
PR = TRASH + 8
NEG = -1e30

def _lane_splat(vec, r):
    idx = jnp.full((16, 1), r, jnp.int32)
    return lax.gather(
        vec, idx,
        lax.GatherDimensionNumbers(offset_dims=(), collapsed_slice_dims=(0,),
                                   start_index_map=(0,)),
        (1,), mode=lax.GatherScatterMode.PROMISE_IN_BOUNDS)


_mesh = plsc.VectorSubcoreMesh(
    core_axis_name="c", subcore_axis_name="s", num_cores=NC, num_subcores=NS)


@functools.partial(
    pl.kernel,
    out_type=jax.ShapeDtypeStruct((NC * NP,), jnp.float32),
    mesh=_mesh,
    compiler_params=pltpu.CompilerParams(use_tc_tiling_on_sc=False),
    scratch_types=dict(
        deg_sp=pltpu.VMEM_SHARED((NP,), jnp.float32),
        zbuf=pltpu.VMEM((3136,), jnp.float32),
        ones=pltpu.VMEM((128,), jnp.float32),
        idx8=pltpu.VMEM((8, 128), jnp.int32),
        sem=pltpu.SemaphoreType.DMA,
    ),
)
def _sc_deg(didx_hbm, degpart_hbm, deg_sp, zbuf, ones, idx8, sem):
    c = lax.axis_index("c")
    s = lax.axis_index("s")
    z16 = jnp.zeros((16,), jnp.float32)
    o16 = jnp.ones((16,), jnp.float32)

    @pl.loop(0, 196)
    def _zb(i):
        zbuf[pl.ds(i * 16, 16)] = z16

    @pl.loop(0, 8)
    def _ob(i):
        ones[pl.ds(i * 16, 16)] = o16

    pltpu.sync_copy(zbuf, deg_sp.at[pl.ds(s * 3136, 3136)])
    plsc.subcore_barrier()

    row0 = c * 3136 + s * 200
    trips = jnp.where(s == 15, 17, 25)

    @pl.loop(0, trips)
    def _chunk(j):
        pltpu.sync_copy(didx_hbm.at[pl.ds(row0 + j * 8, 8), :], idx8)
        cps = [pltpu.async_copy(ones, deg_sp.at[idx8.at[jj]], sem, add=True)
               for jj in range(8)]
        for cp in cps:
            cp.wait()

    plsc.subcore_barrier()
    pltpu.sync_copy(deg_sp.at[pl.ds(s * 3136, 3136)], zbuf)
    pltpu.sync_copy(zbuf, degpart_hbm.at[pl.ds(c * NP + s * 3136, 3136)])


@functools.partial(
    pl.kernel,
    out_type=jax.ShapeDtypeStruct((NC, NP, HH), jnp.float32),
    mesh=_mesh,
    compiler_params=pltpu.CompilerParams(use_tc_tiling_on_sc=False),
    scratch_types=dict(
        acc_sp=pltpu.VMEM_SHARED((NP, HH), jnp.float32),
        rows=pltpu.VMEM((512, HH), jnp.float32),
        sidx=pltpu.VMEM((4, 128), jnp.int32),
        didx=pltpu.VMEM((4, 128), jnp.int32),
        gsem=pltpu.SemaphoreType.DMA,
        ssem=pltpu.SemaphoreType.DMA,
    ),
)
def _sc_prop(hs_hbm, srcoff_hbm, didx_hbm, acc_hbm,
             acc_sp, rows, sidx, didx, gsem, ssem):
    c = lax.axis_index("c")
    s = lax.axis_index("s")
    z16 = jnp.zeros((16,), jnp.float32)

    @pl.loop(0, 512)
    def _zr(i):
        rows[i, pl.ds(0, 16)] = z16
        rows[i, pl.ds(16, 16)] = z16

    slab = s * 3136
    for k in range(6):
        pltpu.sync_copy(rows, acc_sp.at[pl.ds(slab + k * 512, 512), :])
    pltpu.sync_copy(rows.at[pl.ds(0, 64), :],
                    acc_sp.at[pl.ds(slab + 3072, 64), :])
    plsc.subcore_barrier()

    row0 = s * 392

    @pl.loop(0, 98)
    def _chunk(j):
        r0 = row0 + j * 4
        pltpu.sync_copy(srcoff_hbm.at[c, pl.ds(r0, 4), :], sidx)
        pltpu.sync_copy(didx_hbm.at[pl.ds(r0, 4), :], didx)
        gcps = [pltpu.async_copy(hs_hbm.at[sidx.at[jj]],
                                 rows.at[pl.ds(jj * 128, 128), :], gsem)
                for jj in range(4)]
        for cp in gcps:
            cp.wait()
        scps = [pltpu.async_copy(rows.at[pl.ds(jj * 128, 128), :],
                                 acc_sp.at[didx.at[jj]], ssem, add=True)
                for jj in range(4)]
        for cp in scps:
            cp.wait()

    plsc.subcore_barrier()
    pltpu.sync_copy(acc_sp.at[pl.ds(slab, 3136), :],
                    acc_hbm.at[c, pl.ds(slab, 3136), :])


@functools.partial(
    pl.kernel,
    out_type=jax.ShapeDtypeStruct((G, H), jnp.float32),
    mesh=_mesh,
    compiler_params=pltpu.CompilerParams(use_tc_tiling_on_sc=False),
    scratch_types=dict(
        pool_sp=pltpu.VMEM_SHARED((PR, H), jnp.float32),
        negbuf=pltpu.VMEM((104, H), jnp.float32),
        rb=pltpu.VMEM((112, H), jnp.float32),
        pb=pltpu.VMEM((112,), jnp.float32),
        tb=pltpu.VMEM((112,), jnp.int32),
        sbuf=pltpu.VMEM((16, H), jnp.float32),
        mb=pltpu.VMEM((NS * 16, H), jnp.float32),
        ob=pltpu.VMEM((16, H), jnp.float32),
        ssem=pltpu.SemaphoreType.DMA,
    ),
)
def _sc_segmax(x4_hbm, pen_hbm, tgt_hbm, pooled_hbm,
               pool_sp, negbuf, rb, pb, tb, sbuf, mb, ob, ssem):
    c = lax.axis_index("c")
    s = lax.axis_index("s")
    neg16 = jnp.full((16,), NEG, jnp.float32)

    @pl.loop(0, 104)
    def _nb(i):
        for f in range(4):
            negbuf[i, pl.ds(f * 16, 16)] = neg16

    for k in range(5):
        pltpu.sync_copy(negbuf, pool_sp.at[pl.ds(s * TBS + k * 104, 104), :])
    plsc.subcore_barrier()

    base = s * RPT

    def chunk(cc, carry):
        rowbase = base + cc * 112
        pltpu.sync_copy(x4_hbm.at[pl.ds(rowbase, 112), :], rb)
        pltpu.sync_copy(pen_hbm.at[pl.ds(rowbase, 112)], pb)
        pltpu.sync_copy(tgt_hbm.at[pl.ds(rowbase, 112)], tb)

        def group(gi, carry_g):
            pv = pb[pl.ds(gi * 16, 16)]
            tgtv = tb[pl.ds(gi * 16, 16)]
            cvecs = list(carry_g)
            for r in range(16):
                ps = _lane_splat(pv, r)
                row = gi * 16 + r
                for f in range(4):
                    v = rb[row, pl.ds(f * 16, 16)]
                    cvecs[f] = jnp.maximum(cvecs[f] + ps, v)
                    sbuf[r, pl.ds(f * 16, 16)] = cvecs[f]
            pltpu.async_copy(sbuf, pool_sp.at[tgtv], ssem).wait()
            return tuple(cvecs)

        return pl.loop(0, 7, init_carry=carry)(group)

    pl.loop(0, 28, init_carry=(neg16, neg16, neg16, neg16))(chunk)

    plsc.subcore_barrier()

    g0 = c * 256 + s * 16
    for t in range(NS):
        pltpu.sync_copy(pool_sp.at[pl.ds(t * TBS + g0, 16), :],
                        mb.at[pl.ds(t * 16, 16), :])

    @pl.loop(0, 16)
    def _merge(gi):
        for f in range(4):
            m = jnp.full((16,), NEG, jnp.float32)
            for t in range(NS):
                m = jnp.maximum(m, mb[t * 16 + gi, pl.ds(f * 16, 16)])
            ob[gi, pl.ds(f * 16, 16)] = m

    pltpu.sync_copy(ob, pooled_hbm.at[pl.ds(g0, 16), :])


def _tc_a_body(x_ref, degpart_ref, w1_ref, src_ref,
               hs_ref, dis_ref, srcoff_ref):
    deg = degpart_ref[0] + degpart_ref[1] + 1.0
    dis = lax.rsqrt(deg)[:, None]
    t = jnp.dot(x_ref[...], w1_ref[...],
                preferred_element_type=jnp.float32)
    hs = dis * t
    hs_ref[0] = hs[:, :HH]
    hs_ref[1] = hs[:, HH:]
    dis_ref[...] = dis
    srcoff_ref[0] = src_ref[...]
    srcoff_ref[1] = src_ref[...] + NP


def _tc_a(x_pad, degpart, w1p, src3d):
    return pl.pallas_call(
        _tc_a_body,
        grid=(98,),
        in_specs=[
            pl.BlockSpec((512, 8), lambda j: (j, 0)),
            pl.BlockSpec((2, 512), lambda j: (0, j)),
            pl.BlockSpec((8, H), lambda j: (0, 0)),
            pl.BlockSpec((1, 1, 8192), lambda j: (j, 0, 0)),
        ],
        out_specs=[
            pl.BlockSpec((2, 512, HH), lambda j: (0, j, 0)),
            pl.BlockSpec((512, 1), lambda j: (j, 0)),
            pl.BlockSpec((2, 1, 1, 8192), lambda j: (0, j, 0, 0)),
        ],
        out_shape=[
            jax.ShapeDtypeStruct((2, NP, HH), jnp.float32),
            jax.ShapeDtypeStruct((NP, 1), jnp.float32),
            jax.ShapeDtypeStruct((2, 98, 1, 8192), jnp.int32),
        ],
    )(x_pad, degpart, w1p, src3d)


def _tc_bc_body(acc_ref, hs_ref, dis_ref, b_ref, w_ref, out_ref):
    acc = jnp.concatenate([acc_ref[0], acc_ref[1]], axis=1)
    hs = jnp.concatenate([hs_ref[0], hs_ref[1]], axis=1)
    dis = dis_ref[...]
    x = jnp.maximum(dis * (acc + hs) + b_ref[...], 0.0)
    t = jnp.dot(x, w_ref[...], preferred_element_type=jnp.float32)
    hs_out = dis * t
    out_ref[0] = hs_out[:, :HH]
    out_ref[1] = hs_out[:, HH:]


def _tc_bc(acc, hs, dis, b, w):
    return pl.pallas_call(
        _tc_bc_body,
        grid=(98,),
        in_specs=[
            pl.BlockSpec((2, 512, HH), lambda j: (0, j, 0)),
            pl.BlockSpec((2, 512, HH), lambda j: (0, j, 0)),
            pl.BlockSpec((512, 1), lambda j: (j, 0)),
            pl.BlockSpec((1, H), lambda j: (0, 0)),
            pl.BlockSpec((H, H), lambda j: (0, 0)),
        ],
        out_specs=pl.BlockSpec((2, 512, HH), lambda j: (0, j, 0)),
        out_shape=jax.ShapeDtypeStruct((2, NP, HH), jnp.float32),
    )(acc, hs, dis, b, w)


def _tc_d_body(acc_ref, hs_ref, dis_ref, b_ref, bat_ref, bprev_ref,
               bnext_ref, x4_ref, pen_ref, tgt_ref):
    acc = jnp.concatenate([acc_ref[0], acc_ref[1]], axis=1)
    hs = jnp.concatenate([hs_ref[0], hs_ref[1]], axis=1)
    dis = dis_ref[...]
    x4 = jnp.maximum(dis * (acc + hs) + b_ref[...], 0.0)
    x4_ref[...] = x4

    j = pl.program_id(0)
    rid = j * 512 + lax.broadcasted_iota(jnp.int32, (512, 1), 0)
    bat = bat_ref[...]
    rmod = rid % RPT
    is_start = (bat != bprev_ref[...]) | (rmod == 0)
    is_end = (bat != bnext_ref[...]) | (rmod == RPT - 1)
    pen_ref[...] = jnp.where(is_start, NEG, 0.0).astype(jnp.float32)
    wk = rid // RPT
    tgt_ref[...] = jnp.where(is_end, wk * TBS + bat, TRASH).astype(jnp.int32)


def _tc_d(acc, hs, dis, b, bat, bprev, bnext):
    return pl.pallas_call(
        _tc_d_body,
        grid=(98,),
        in_specs=[
            pl.BlockSpec((2, 512, HH), lambda j: (0, j, 0)),
            pl.BlockSpec((2, 512, HH), lambda j: (0, j, 0)),
            pl.BlockSpec((512, 1), lambda j: (j, 0)),
            pl.BlockSpec((1, H), lambda j: (0, 0)),
            pl.BlockSpec((512, 1), lambda j: (j, 0)),
            pl.BlockSpec((512, 1), lambda j: (j, 0)),
            pl.BlockSpec((512, 1), lambda j: (j, 0)),
        ],
        out_specs=[
            pl.BlockSpec((512, H), lambda j: (j, 0)),
            pl.BlockSpec((512, 1), lambda j: (j, 0)),
            pl.BlockSpec((512, 1), lambda j: (j, 0)),
        ],
        out_shape=[
            jax.ShapeDtypeStruct((NP, H), jnp.float32),
            jax.ShapeDtypeStruct((NP, 1), jnp.float32),
            jax.ShapeDtypeStruct((NP, 1), jnp.int32),
        ],
    )(acc, hs, dis, b, bat, bprev, bnext)


def _tc_e_body(pooled_ref, wl_ref, bl_ref, out_ref):
    out_ref[...] = jnp.dot(pooled_ref[...], wl_ref[...],
                           preferred_element_type=jnp.float32) + bl_ref[...]


def _tc_e(pooled, wl, bl2d):
    return pl.pallas_call(
        _tc_e_body,
        out_shape=jax.ShapeDtypeStruct((G, 2), jnp.float32),
    )(pooled, wl, bl2d)


def kernel(x, edge_index, batch_data, W1, b1, W2, b2, W3, b3, Wl, bl):
    src = edge_index[0]
    dst = edge_index[1]
    src_pad = jnp.pad(src, (0, EP - E))
    dst_pad = jnp.pad(dst, (0, EP - E), constant_values=N)
    src3d = src_pad.reshape(98, 1, 8192)
    didx2d = dst_pad.reshape(ER, 128)

    x_pad = jnp.pad(x, ((0, NP - N), (0, 1)))
    w1p = jnp.pad(W1, ((0, 1), (0, 0)))

    batf = jnp.pad(batch_data, (0, NP - N), constant_values=G)
    bat = batf.reshape(NP, 1)
    bprev = jnp.concatenate(
        [jnp.full((1,), -1, jnp.int32), batf[:NP - 1]]).reshape(NP, 1)
    bnext = jnp.concatenate(
        [batf[1:], jnp.full((1,), G + 1, jnp.int32)]).reshape(NP, 1)

    degpart = _sc_deg(didx2d).reshape(NC, NP)
    hs1, dis, srcoff = _tc_a(x_pad, degpart, w1p, src3d)
    srcoff3d = srcoff.reshape(NC, ER, 128)

    b1r = b1.reshape(1, H)
    b2r = b2.reshape(1, H)
    b3r = b3.reshape(1, H)

    acc1 = _sc_prop(hs1.reshape(NC * NP, HH), srcoff3d, didx2d)
    hs2 = _tc_bc(acc1, hs1, dis, b1r, W2)
    acc2 = _sc_prop(hs2.reshape(NC * NP, HH), srcoff3d, didx2d)
    hs3 = _tc_bc(acc2, hs2, dis, b2r, W3)
    acc3 = _sc_prop(hs3.reshape(NC * NP, HH), srcoff3d, didx2d)
    x4, pen, tgt = _tc_d(acc3, hs3, dis, b3r, bat, bprev, bnext)

    pooled = _sc_segmax(x4, pen.reshape(NP), tgt.reshape(NP))
    out = _tc_e(pooled, Wl, bl.reshape(1, 2))
    return out


# speedup vs baseline: 16.6816x; 1.0459x over previous
import functools

import jax
import jax.numpy as jnp
from jax import lax
from jax.experimental import pallas as pl
from jax.experimental.pallas import tpu as pltpu
from jax.experimental.pallas import tpu_sc as plsc

N = 50000
E = 800000
H = 64
HH = 32
G = 512
NC, NS = 2, 16
NW = NC * NS

NP = 50176
EP = 802816
ER = EP // 128
RPT = NP // NS
TBS = 520
TRASH = NS * TBS
PR = TRASH + 8
NEG = -1e30

def _lane_splat(vec, r):
    idx = jnp.full((16, 1), r, jnp.int32)
    return lax.gather(
        vec, idx,
        lax.GatherDimensionNumbers(offset_dims=(), collapsed_slice_dims=(0,),
                                   start_index_map=(0,)),
        (1,), mode=lax.GatherScatterMode.PROMISE_IN_BOUNDS)


_mesh = plsc.VectorSubcoreMesh(
    core_axis_name="c", subcore_axis_name="s", num_cores=NC, num_subcores=NS)


@functools.partial(
    pl.kernel,
    out_type=jax.ShapeDtypeStruct((NC * NP,), jnp.float32),
    mesh=_mesh,
    compiler_params=pltpu.CompilerParams(use_tc_tiling_on_sc=False),
    scratch_types=dict(
        deg_sp=pltpu.VMEM_SHARED((NP,), jnp.float32),
        zbuf=pltpu.VMEM((3136,), jnp.float32),
        ones=pltpu.VMEM((128,), jnp.float32),
        idx8=pltpu.VMEM((8, 128), jnp.int32),
        sem=pltpu.SemaphoreType.DMA,
    ),
)
def _sc_deg(didx_hbm, degpart_hbm, deg_sp, zbuf, ones, idx8, sem):
    c = lax.axis_index("c")
    s = lax.axis_index("s")
    z16 = jnp.zeros((16,), jnp.float32)
    o16 = jnp.ones((16,), jnp.float32)

    @pl.loop(0, 196)
    def _zb(i):
        zbuf[pl.ds(i * 16, 16)] = z16

    @pl.loop(0, 8)
    def _ob(i):
        ones[pl.ds(i * 16, 16)] = o16

    pltpu.sync_copy(zbuf, deg_sp.at[pl.ds(s * 3136, 3136)])
    plsc.subcore_barrier()

    row0 = c * 3136 + s * 200
    trips = jnp.where(s == 15, 17, 25)

    @pl.loop(0, trips)
    def _chunk(j):
        pltpu.sync_copy(didx_hbm.at[pl.ds(row0 + j * 8, 8), :], idx8)
        cps = [pltpu.async_copy(ones, deg_sp.at[idx8.at[jj]], sem, add=True)
               for jj in range(8)]
        for cp in cps:
            cp.wait()

    plsc.subcore_barrier()
    pltpu.sync_copy(deg_sp.at[pl.ds(s * 3136, 3136)], zbuf)
    pltpu.sync_copy(zbuf, degpart_hbm.at[pl.ds(c * NP + s * 3136, 3136)])


@functools.partial(
    pl.kernel,
    out_type=jax.ShapeDtypeStruct((NC, NP, HH), jnp.float32),
    mesh=_mesh,
    compiler_params=pltpu.CompilerParams(use_tc_tiling_on_sc=False),
    scratch_types=dict(
        acc_sp=pltpu.VMEM_SHARED((NP, HH), jnp.float32),
        rows0=pltpu.VMEM((256, HH), jnp.float32),
        rows1=pltpu.VMEM((256, HH), jnp.float32),
        sidx0=pltpu.VMEM((2, 128), jnp.int32),
        sidx1=pltpu.VMEM((2, 128), jnp.int32),
        didx0=pltpu.VMEM((2, 128), jnp.int32),
        didx1=pltpu.VMEM((2, 128), jnp.int32),
        gsem=pltpu.SemaphoreType.DMA,
        ssem=pltpu.SemaphoreType.DMA,
    ),
)
def _sc_prop(hs_hbm, srcoff_hbm, didx_hbm, acc_hbm,
             acc_sp, rows0, rows1, sidx0, sidx1, didx0, didx1, gsem, ssem):
    c = lax.axis_index("c")
    s = lax.axis_index("s")
    z16 = jnp.zeros((16,), jnp.float32)
    rows = (rows0, rows1)
    sidx = (sidx0, sidx1)
    didx = (didx0, didx1)

    @pl.loop(0, 256)
    def _zr(i):
        for b in range(2):
            rows[b][i, pl.ds(0, 16)] = z16
            rows[b][i, pl.ds(16, 16)] = z16

    slab = s * 3136
    for k in range(12):
        pltpu.sync_copy(rows[k % 2], acc_sp.at[pl.ds(slab + k * 256, 256), :])
    pltpu.sync_copy(rows0.at[pl.ds(0, 64), :],
                    acc_sp.at[pl.ds(slab + 3072, 64), :])
    plsc.subcore_barrier()

    row0_ = s * 392

    def _load_idx(j, b):
        r0 = row0_ + j * 2
        pltpu.sync_copy(srcoff_hbm.at[c, pl.ds(r0, 2), :], sidx[b])
        pltpu.sync_copy(didx_hbm.at[pl.ds(r0, 2), :], didx[b])

    def _fire_gather(b):
        return [pltpu.async_copy(hs_hbm.at[sidx[b].at[jj]],
                                 rows[b].at[pl.ds(jj * 128, 128), :], gsem)
                for jj in range(2)]

    def _fire_scatter(b):
        return [pltpu.async_copy(rows[b].at[pl.ds(jj * 128, 128), :],
                                 acc_sp.at[didx[b].at[jj]], ssem, add=True)
                for jj in range(2)]

    _load_idx(0, 0)

    @pl.loop(0, 196, step=2)
    def _pipe(j):
        for b in range(2):
            k = j + b
            o = 1 - b
            gcps = _fire_gather(b)

            @pl.when(k > 0)
            def _scat():
                scps = _fire_scatter(o)
                for cp in scps:
                    cp.wait()

            @pl.when(k < 195)
            def _pref():
                _load_idx(k + 1, o)

            for cp in gcps:
                cp.wait()

    scps = _fire_scatter(1)
    for cp in scps:
        cp.wait()

    plsc.subcore_barrier()
    pltpu.sync_copy(acc_sp.at[pl.ds(slab, 3136), :],
                    acc_hbm.at[c, pl.ds(slab, 3136), :])


@functools.partial(
    pl.kernel,
    out_type=jax.ShapeDtypeStruct((G, H), jnp.float32),
    mesh=_mesh,
    compiler_params=pltpu.CompilerParams(use_tc_tiling_on_sc=False),
    scratch_types=dict(
        pool_sp=pltpu.VMEM_SHARED((PR, H), jnp.float32),
        negbuf=pltpu.VMEM((104, H), jnp.float32),
        rb=pltpu.VMEM((112, H), jnp.float32),
        pb=pltpu.VMEM((112,), jnp.float32),
        tb=pltpu.VMEM((112,), jnp.int32),
        sbuf=pltpu.VMEM((16, H), jnp.float32),
        mb=pltpu.VMEM((NS * 16, H), jnp.float32),
        ob=pltpu.VMEM((16, H), jnp.float32),
        ssem=pltpu.SemaphoreType.DMA,
    ),
)
def _sc_segmax(x4_hbm, pen_hbm, tgt_hbm, pooled_hbm,
               pool_sp, negbuf, rb, pb, tb, sbuf, mb, ob, ssem):
    c = lax.axis_index("c")
    s = lax.axis_index("s")
    neg16 = jnp.full((16,), NEG, jnp.float32)

    @pl.loop(0, 104)
    def _nb(i):
        for f in range(4):
            negbuf[i, pl.ds(f * 16, 16)] = neg16

    for k in range(5):
        pltpu.sync_copy(negbuf, pool_sp.at[pl.ds(s * TBS + k * 104, 104), :])
    plsc.subcore_barrier()

    base = s * RPT

    def chunk(cc, carry):
        rowbase = base + cc * 112
        pltpu.sync_copy(x4_hbm.at[pl.ds(rowbase, 112), :], rb)
        pltpu.sync_copy(pen_hbm.at[pl.ds(rowbase, 112)], pb)
        pltpu.sync_copy(tgt_hbm.at[pl.ds(rowbase, 112)], tb)

        def group(gi, carry_g):
            pv = pb[pl.ds(gi * 16, 16)]
            tgtv = tb[pl.ds(gi * 16, 16)]
            cvecs = list(carry_g)
            for r in range(16):
                ps = _lane_splat(pv, r)
                row = gi * 16 + r
                for f in range(4):
                    v = rb[row, pl.ds(f * 16, 16)]
                    cvecs[f] = jnp.maximum(cvecs[f] + ps, v)
                    sbuf[r, pl.ds(f * 16, 16)] = cvecs[f]
            pltpu.async_copy(sbuf, pool_sp.at[tgtv], ssem).wait()
            return tuple(cvecs)

        return pl.loop(0, 7, init_carry=carry)(group)

    pl.loop(0, 28, init_carry=(neg16, neg16, neg16, neg16))(chunk)

    plsc.subcore_barrier()

    g0 = c * 256 + s * 16
    for t in range(NS):
        pltpu.sync_copy(pool_sp.at[pl.ds(t * TBS + g0, 16), :],
                        mb.at[pl.ds(t * 16, 16), :])

    @pl.loop(0, 16)
    def _merge(gi):
        for f in range(4):
            m = jnp.full((16,), NEG, jnp.float32)
            for t in range(NS):
                m = jnp.maximum(m, mb[t * 16 + gi, pl.ds(f * 16, 16)])
            ob[gi, pl.ds(f * 16, 16)] = m

    pltpu.sync_copy(ob, pooled_hbm.at[pl.ds(g0, 16), :])


def _tc_a_body(x_ref, degpart_ref, w1_ref, src_ref,
               hs_ref, dis_ref, srcoff_ref):
    deg = degpart_ref[0] + degpart_ref[1] + 1.0
    dis = lax.rsqrt(deg)[:, None]
    t = jnp.dot(x_ref[...], w1_ref[...],
                preferred_element_type=jnp.float32)
    hs = dis * t
    hs_ref[0] = hs[:, :HH]
    hs_ref[1] = hs[:, HH:]
    dis_ref[...] = dis
    srcoff_ref[0] = src_ref[...]
    srcoff_ref[1] = src_ref[...] + NP


def _tc_a(x_pad, degpart, w1p, src3d):
    return pl.pallas_call(
        _tc_a_body,
        grid=(98,),
        in_specs=[
            pl.BlockSpec((512, 8), lambda j: (j, 0)),
            pl.BlockSpec((2, 512), lambda j: (0, j)),
            pl.BlockSpec((8, H), lambda j: (0, 0)),
            pl.BlockSpec((1, 1, 8192), lambda j: (j, 0, 0)),
        ],
        out_specs=[
            pl.BlockSpec((2, 512, HH), lambda j: (0, j, 0)),
            pl.BlockSpec((512, 1), lambda j: (j, 0)),
            pl.BlockSpec((2, 1, 1, 8192), lambda j: (0, j, 0, 0)),
        ],
        out_shape=[
            jax.ShapeDtypeStruct((2, NP, HH), jnp.float32),
            jax.ShapeDtypeStruct((NP, 1), jnp.float32),
            jax.ShapeDtypeStruct((2, 98, 1, 8192), jnp.int32),
        ],
    )(x_pad, degpart, w1p, src3d)


def _tc_bc_body(acc_ref, hs_ref, dis_ref, b_ref, w_ref, out_ref):
    acc = jnp.concatenate([acc_ref[0], acc_ref[1]], axis=1)
    hs = jnp.concatenate([hs_ref[0], hs_ref[1]], axis=1)
    dis = dis_ref[...]
    x = jnp.maximum(dis * (acc + hs) + b_ref[...], 0.0)
    t = jnp.dot(x, w_ref[...], preferred_element_type=jnp.float32)
    hs_out = dis * t
    out_ref[0] = hs_out[:, :HH]
    out_ref[1] = hs_out[:, HH:]


def _tc_bc(acc, hs, dis, b, w):
    return pl.pallas_call(
        _tc_bc_body,
        grid=(98,),
        in_specs=[
            pl.BlockSpec((2, 512, HH), lambda j: (0, j, 0)),
            pl.BlockSpec((2, 512, HH), lambda j: (0, j, 0)),
            pl.BlockSpec((512, 1), lambda j: (j, 0)),
            pl.BlockSpec((1, H), lambda j: (0, 0)),
            pl.BlockSpec((H, H), lambda j: (0, 0)),
        ],
        out_specs=pl.BlockSpec((2, 512, HH), lambda j: (0, j, 0)),
        out_shape=jax.ShapeDtypeStruct((2, NP, HH), jnp.float32),
    )(acc, hs, dis, b, w)


def _tc_d_body(acc_ref, hs_ref, dis_ref, b_ref, bat_ref, bprev_ref,
               bnext_ref, x4_ref, pen_ref, tgt_ref):
    acc = jnp.concatenate([acc_ref[0], acc_ref[1]], axis=1)
    hs = jnp.concatenate([hs_ref[0], hs_ref[1]], axis=1)
    dis = dis_ref[...]
    x4 = jnp.maximum(dis * (acc + hs) + b_ref[...], 0.0)
    x4_ref[...] = x4

    j = pl.program_id(0)
    rid = j * 512 + lax.broadcasted_iota(jnp.int32, (512, 1), 0)
    bat = bat_ref[...]
    rmod = rid % RPT
    is_start = (bat != bprev_ref[...]) | (rmod == 0)
    is_end = (bat != bnext_ref[...]) | (rmod == RPT - 1)
    pen_ref[...] = jnp.where(is_start, NEG, 0.0).astype(jnp.float32)
    wk = rid // RPT
    tgt_ref[...] = jnp.where(is_end, wk * TBS + bat, TRASH).astype(jnp.int32)


def _tc_d(acc, hs, dis, b, bat, bprev, bnext):
    return pl.pallas_call(
        _tc_d_body,
        grid=(98,),
        in_specs=[
            pl.BlockSpec((2, 512, HH), lambda j: (0, j, 0)),
            pl.BlockSpec((2, 512, HH), lambda j: (0, j, 0)),
            pl.BlockSpec((512, 1), lambda j: (j, 0)),
            pl.BlockSpec((1, H), lambda j: (0, 0)),
            pl.BlockSpec((512, 1), lambda j: (j, 0)),
            pl.BlockSpec((512, 1), lambda j: (j, 0)),
            pl.BlockSpec((512, 1), lambda j: (j, 0)),
        ],
        out_specs=[
            pl.BlockSpec((512, H), lambda j: (j, 0)),
            pl.BlockSpec((512, 1), lambda j: (j, 0)),
            pl.BlockSpec((512, 1), lambda j: (j, 0)),
        ],
        out_shape=[
            jax.ShapeDtypeStruct((NP, H), jnp.float32),
            jax.ShapeDtypeStruct((NP, 1), jnp.float32),
            jax.ShapeDtypeStruct((NP, 1), jnp.int32),
        ],
    )(acc, hs, dis, b, bat, bprev, bnext)


def _tc_e_body(pooled_ref, wl_ref, bl_ref, out_ref):
    out_ref[...] = jnp.dot(pooled_ref[...], wl_ref[...],
                           preferred_element_type=jnp.float32) + bl_ref[...]


def _tc_e(pooled, wl, bl2d):
    return pl.pallas_call(
        _tc_e_body,
        out_shape=jax.ShapeDtypeStruct((G, 2), jnp.float32),
    )(pooled, wl, bl2d)


def kernel(x, edge_index, batch_data, W1, b1, W2, b2, W3, b3, Wl, bl):
    src = edge_index[0]
    dst = edge_index[1]
    src_pad = jnp.pad(src, (0, EP - E))
    dst_pad = jnp.pad(dst, (0, EP - E), constant_values=N)
    src3d = src_pad.reshape(98, 1, 8192)
    didx2d = dst_pad.reshape(ER, 128)

    x_pad = jnp.pad(x, ((0, NP - N), (0, 1)))
    w1p = jnp.pad(W1, ((0, 1), (0, 0)))

    batf = jnp.pad(batch_data, (0, NP - N), constant_values=G)
    bat = batf.reshape(NP, 1)
    bprev = jnp.concatenate(
        [jnp.full((1,), -1, jnp.int32), batf[:NP - 1]]).reshape(NP, 1)
    bnext = jnp.concatenate(
        [batf[1:], jnp.full((1,), G + 1, jnp.int32)]).reshape(NP, 1)

    degpart = _sc_deg(didx2d).reshape(NC, NP)
    hs1, dis, srcoff = _tc_a(x_pad, degpart, w1p, src3d)
    srcoff3d = srcoff.reshape(NC, ER, 128)

    b1r = b1.reshape(1, H)
    b2r = b2.reshape(1, H)
    b3r = b3.reshape(1, H)

    acc1 = _sc_prop(hs1.reshape(NC * NP, HH), srcoff3d, didx2d)
    hs2 = _tc_bc(acc1, hs1, dis, b1r, W2)
    acc2 = _sc_prop(hs2.reshape(NC * NP, HH), srcoff3d, didx2d)
    hs3 = _tc_bc(acc2, hs2, dis, b2r, W3)
    acc3 = _sc_prop(hs3.reshape(NC * NP, HH), srcoff3d, didx2d)
    x4, pen, tgt = _tc_d(acc3, hs3, dis, b3r, bat, bprev, bnext)

    pooled = _sc_segmax(x4, pen.reshape(NP), tgt.reshape(NP))
    out = _tc_e(pooled, Wl, bl.reshape(1, 2))
    return out
